# baseline (speedup 1.0000x reference)
import jax
import jax.numpy as jnp
import numpy as np
from jax.experimental import pallas as pl

B, L, N, C = 8, 12, 50000, 3
EDIM = 32

BLOCK_N = 2048
NUM_NB = -(-N // BLOCK_N)
VW = 128

def _stid_kernel(x_ref, wr_ref, b_ref, node_ref, tidT_ref, diwT_ref, out_ref):
    xb = x_ref[0]

    k_iota = jax.lax.broadcasted_iota(jnp.int32, (L, VW), 1)
    xc = []
    for c in range(C):
        pos = 3 * k_iota + c
        m_vec = pos >> 7
        r_idx = pos & 127
        parts = []
        for j in range(BLOCK_N // VW):
            base = 3 * VW * j
            g0 = jnp.take_along_axis(xb[:, base:base + VW], r_idx, axis=1)
            g1 = jnp.take_along_axis(xb[:, base + VW:base + 2 * VW], r_idx, axis=1)
            g2 = jnp.take_along_axis(xb[:, base + 2 * VW:base + 3 * VW], r_idx, axis=1)
            parts.append(jnp.where(m_vec == 0, g0, jnp.where(m_vec == 1, g1, g2)))
        xc.append(jnp.concatenate(parts, axis=1))

    dn = (((1,), (0,)), ((), ()))
    ts = b_ref[...]
    for c in range(C):
        ts = ts + jax.lax.dot_general(
            wr_ref[c], xc[c], dn, preferred_element_type=jnp.float32)

    tid_idx = jnp.clip((xc[1][L - 1:L, :] * 288.0).astype(jnp.int32), 0, 287)
    diw_idx = jnp.clip((xc[2][L - 1:L, :] * 7.0).astype(jnp.int32), 0, 6)

    c0 = tidT_ref[:, 0:VW]
    c1 = tidT_ref[:, VW:2 * VW]
    c2 = tidT_ref[:, 2 * VW:3 * VW]
    dw = diwT_ref[...]
    tid_parts = []
    diw_parts = []
    for j in range(BLOCK_N // VW):
        tj = jnp.broadcast_to(tid_idx[:, VW * j:VW * (j + 1)], (EDIM, VW))
        g0 = jnp.take_along_axis(c0, jnp.clip(tj, 0, VW - 1), axis=1)
        g1 = jnp.take_along_axis(c1, jnp.clip(tj - VW, 0, VW - 1), axis=1)
        g2 = jnp.take_along_axis(c2, jnp.clip(tj - 2 * VW, 0, VW - 1), axis=1)
        tid_parts.append(jnp.where(tj < VW, g0, jnp.where(tj < 2 * VW, g1, g2)))
        dj = jnp.broadcast_to(diw_idx[:, VW * j:VW * (j + 1)], (EDIM, VW))
        diw_parts.append(jnp.take_along_axis(dw, dj, axis=1))
    tid = jnp.concatenate(tid_parts, axis=1)
    diw = jnp.concatenate(diw_parts, axis=1)

    out_ref[0, 0:32, 0, :] = ts
    out_ref[0, 32:64, 0, :] = node_ref[...]
    out_ref[0, 64:96, 0, :] = tid
    out_ref[0, 96:128, 0, :] = diw


def kernel(x, node_emb, time_in_day_emb, day_in_week_emb, W, b):
    xr = x.reshape(B, L, N * C)
    Wr = W.reshape(EDIM, L, C).transpose(2, 0, 1)
    nodeT = node_emb.T
    tidT = jnp.pad(time_in_day_emb.T, ((0, 0), (0, 384 - 288)))
    diwT = jnp.pad(day_in_week_emb.T, ((0, 0), (0, 128 - 7)))
    b2 = b.reshape(EDIM, 1)

    out = pl.pallas_call(
        _stid_kernel,
        grid=(NUM_NB, B),
        in_specs=[
            pl.BlockSpec((1, L, C * BLOCK_N), lambda nb, bb: (bb, 0, nb)),
            pl.BlockSpec((C, EDIM, L), lambda nb, bb: (0, 0, 0)),
            pl.BlockSpec((EDIM, 1), lambda nb, bb: (0, 0)),
            pl.BlockSpec((EDIM, BLOCK_N), lambda nb, bb: (0, nb)),
            pl.BlockSpec((EDIM, 384), lambda nb, bb: (0, 0)),
            pl.BlockSpec((EDIM, 128), lambda nb, bb: (0, 0)),
        ],
        out_specs=pl.BlockSpec((1, 4 * EDIM, 1, BLOCK_N),
                               lambda nb, bb: (bb, 0, 0, nb)),
        out_shape=jax.ShapeDtypeStruct((B, 4 * EDIM, 1, N), jnp.float32),
    )(xr, Wr, b2, nodeT, tidT, diwT)
    return out.reshape(B, 4 * EDIM, N, 1)


# speedup vs baseline: 4.1717x; 4.1717x over previous
import jax
import jax.numpy as jnp
from jax.experimental import pallas as pl

B, L, N, C = 8, 12, 50000, 3
EDIM = 32
R = L * C * B

BLOCK_N = 2048
NUM_NB = -(-N // BLOCK_N)
VW = 128


def _stid_kernel(xf_ref, wb_ref, b_ref, node_ref, tidT_ref, diwT_ref, out_ref):
    bb = pl.program_id(1)
    xfb = xf_ref[...]

    ts = jax.lax.dot_general(
        wb_ref[0], xfb, (((1,), (0,)), ((), ())),
        preferred_element_type=jnp.float32)
    ts = ts + b_ref[...]

    tvals = xf_ref[pl.ds((L - 1) * 24 + 8 + bb, 1), :]
    dvals = xf_ref[pl.ds((L - 1) * 24 + 16 + bb, 1), :]
    tid_idx = jnp.clip((tvals * 288.0).astype(jnp.int32), 0, 287)
    diw_idx = jnp.clip((dvals * 7.0).astype(jnp.int32), 0, 6)

    c0 = tidT_ref[:, 0:VW]
    c1 = tidT_ref[:, VW:2 * VW]
    c2 = tidT_ref[:, 2 * VW:288]
    dw = diwT_ref[...]
    tid_parts = []
    diw_parts = []
    for j in range(BLOCK_N // VW):
        tj = jnp.broadcast_to(tid_idx[:, VW * j:VW * (j + 1)], (EDIM, VW))
        g0 = jnp.take_along_axis(c0, jnp.clip(tj, 0, VW - 1), axis=1)
        g1 = jnp.take_along_axis(c1, jnp.clip(tj - VW, 0, VW - 1), axis=1)
        g2 = jnp.take_along_axis(c2, jnp.clip(tj - 2 * VW, 0, 31), axis=1)
        tid_parts.append(jnp.where(tj < VW, g0, jnp.where(tj < 2 * VW, g1, g2)))
        dj = jnp.broadcast_to(diw_idx[:, VW * j:VW * (j + 1)], (EDIM, VW))
        diw_parts.append(jnp.take_along_axis(dw, dj, axis=1))
    tid = jnp.concatenate(tid_parts, axis=1)
    diw = jnp.concatenate(diw_parts, axis=1)

    h = jnp.concatenate([ts, node_ref[...], tid, diw], axis=0)
    out_ref[0] = jnp.transpose(h)


def kernel(x, node_emb, time_in_day_emb, day_in_week_emb, W, b):
    xf = jnp.transpose(x, (1, 3, 0, 2)).reshape(R, N)
    nodeT = node_emb.T
    tidT = time_in_day_emb.T
    diwT = day_in_week_emb.T
    b2 = b.reshape(EDIM, 1)
    W3 = W.reshape(EDIM, L, C)
    Wbig = (W3[None, :, :, :, None] *
            jnp.eye(B, dtype=W.dtype)[:, None, None, None, :]).reshape(B, EDIM, R)

    out = pl.pallas_call(
        _stid_kernel,
        grid=(NUM_NB, B),
        in_specs=[
            pl.BlockSpec((R, BLOCK_N), lambda nb, bb: (0, nb)),
            pl.BlockSpec((1, EDIM, R), lambda nb, bb: (bb, 0, 0)),
            pl.BlockSpec((EDIM, 1), lambda nb, bb: (0, 0)),
            pl.BlockSpec((EDIM, BLOCK_N), lambda nb, bb: (0, nb)),
            pl.BlockSpec((EDIM, 288), lambda nb, bb: (0, 0)),
            pl.BlockSpec((EDIM, 7), lambda nb, bb: (0, 0)),
        ],
        out_specs=pl.BlockSpec((1, BLOCK_N, 4 * EDIM),
                               lambda nb, bb: (bb, nb, 0)),
        out_shape=jax.ShapeDtypeStruct((B, N, 4 * EDIM), jnp.float32),
    )(xf, Wbig, b2, nodeT, tidT, diwT)
    return jnp.transpose(out, (0, 2, 1))[..., None]


# speedup vs baseline: 4.6475x; 1.1140x over previous
import jax
import jax.numpy as jnp
from jax.experimental import pallas as pl

B, L, N, C = 8, 12, 50000, 3
EDIM = 32
R = L * C * B

BLOCK_N = 4096
NUM_NB = -(-N // BLOCK_N)
VW = 128


def _stid_kernel(xf_ref, wb_ref, b_ref, node_ref, tidT_ref, diwT_ref, out_ref):
    bb = pl.program_id(1)
    xfb = xf_ref[...]

    ts = jax.lax.dot_general(
        wb_ref[0], xfb, (((1,), (0,)), ((), ())),
        preferred_element_type=jnp.float32)
    ts = ts + b_ref[...]

    tvals = xf_ref[pl.ds((L - 1) * 24 + 8 + bb, 1), :]
    dvals = xf_ref[pl.ds((L - 1) * 24 + 16 + bb, 1), :]
    tid_idx = jnp.clip((tvals * 288.0).astype(jnp.int32), 0, 287)
    diw_idx = jnp.clip((dvals * 7.0).astype(jnp.int32), 0, 6)

    c0 = tidT_ref[:, 0:VW]
    c1 = tidT_ref[:, VW:2 * VW]
    c2 = tidT_ref[:, 2 * VW:288]
    dw = diwT_ref[...]
    tid_parts = []
    diw_parts = []
    for j in range(BLOCK_N // VW):
        tj = jnp.broadcast_to(tid_idx[:, VW * j:VW * (j + 1)], (EDIM, VW))
        g0 = jnp.take_along_axis(c0, jnp.clip(tj, 0, VW - 1), axis=1)
        g1 = jnp.take_along_axis(c1, jnp.clip(tj - VW, 0, VW - 1), axis=1)
        g2 = jnp.take_along_axis(c2, jnp.clip(tj - 2 * VW, 0, 31), axis=1)
        tid_parts.append(jnp.where(tj < VW, g0, jnp.where(tj < 2 * VW, g1, g2)))
        dj = jnp.broadcast_to(diw_idx[:, VW * j:VW * (j + 1)], (EDIM, VW))
        diw_parts.append(jnp.take_along_axis(dw, dj, axis=1))
    tid = jnp.concatenate(tid_parts, axis=1)
    diw = jnp.concatenate(diw_parts, axis=1)

    h = jnp.concatenate([ts, node_ref[...], tid, diw], axis=0)
    out_ref[0] = jnp.transpose(h)


def kernel(x, node_emb, time_in_day_emb, day_in_week_emb, W, b):
    xf = jnp.transpose(x, (1, 3, 0, 2)).reshape(R, N)
    nodeT = node_emb.T
    tidT = time_in_day_emb.T
    diwT = day_in_week_emb.T
    b2 = b.reshape(EDIM, 1)
    W3 = W.reshape(EDIM, L, C)
    Wbig = (W3[None, :, :, :, None] *
            jnp.eye(B, dtype=W.dtype)[:, None, None, None, :]).reshape(B, EDIM, R)

    out = pl.pallas_call(
        _stid_kernel,
        grid=(NUM_NB, B),
        in_specs=[
            pl.BlockSpec((R, BLOCK_N), lambda nb, bb: (0, nb)),
            pl.BlockSpec((1, EDIM, R), lambda nb, bb: (bb, 0, 0)),
            pl.BlockSpec((EDIM, 1), lambda nb, bb: (0, 0)),
            pl.BlockSpec((EDIM, BLOCK_N), lambda nb, bb: (0, nb)),
            pl.BlockSpec((EDIM, 288), lambda nb, bb: (0, 0)),
            pl.BlockSpec((EDIM, 7), lambda nb, bb: (0, 0)),
        ],
        out_specs=pl.BlockSpec((1, BLOCK_N, 4 * EDIM),
                               lambda nb, bb: (bb, nb, 0)),
        out_shape=jax.ShapeDtypeStruct((B, N, 4 * EDIM), jnp.float32),
    )(xf, Wbig, b2, nodeT, tidT, diwT)
    return jnp.transpose(out, (0, 2, 1))[..., None]


# speedup vs baseline: 5.0569x; 1.0881x over previous
import jax
import jax.numpy as jnp
from jax.experimental import pallas as pl

B, L, N, C = 8, 12, 50000, 3
EDIM = 32
R = L * C * B

BLOCK_N = 6400
NUM_NB = -(-N // BLOCK_N)
VW = 128


def _stid_kernel(xf_ref, wb_ref, b_ref, node_ref, tidT_ref, diwT_ref, out_ref):
    bb = pl.program_id(1)
    xfb = xf_ref[...]

    ts = jax.lax.dot_general(
        wb_ref[0], xfb, (((1,), (0,)), ((), ())),
        preferred_element_type=jnp.float32)
    ts = ts + b_ref[...]

    tvals = xf_ref[pl.ds((L - 1) * 24 + 8 + bb, 1), :]
    dvals = xf_ref[pl.ds((L - 1) * 24 + 16 + bb, 1), :]
    tid_idx = jnp.clip((tvals * 288.0).astype(jnp.int32), 0, 287)
    diw_idx = jnp.clip((dvals * 7.0).astype(jnp.int32), 0, 6)

    c0 = tidT_ref[:, 0:VW]
    c1 = tidT_ref[:, VW:2 * VW]
    c2 = tidT_ref[:, 2 * VW:288]
    dw = diwT_ref[...]
    tid_parts = []
    diw_parts = []
    for j in range(BLOCK_N // VW):
        tj = jnp.broadcast_to(tid_idx[:, VW * j:VW * (j + 1)], (EDIM, VW))
        g0 = jnp.take_along_axis(c0, jnp.clip(tj, 0, VW - 1), axis=1)
        g1 = jnp.take_along_axis(c1, jnp.clip(tj - VW, 0, VW - 1), axis=1)
        g2 = jnp.take_along_axis(c2, jnp.clip(tj - 2 * VW, 0, 31), axis=1)
        tid_parts.append(jnp.where(tj < VW, g0, jnp.where(tj < 2 * VW, g1, g2)))
        dj = jnp.broadcast_to(diw_idx[:, VW * j:VW * (j + 1)], (EDIM, VW))
        diw_parts.append(jnp.take_along_axis(dw, dj, axis=1))
    tid = jnp.concatenate(tid_parts, axis=1)
    diw = jnp.concatenate(diw_parts, axis=1)

    h = jnp.concatenate([ts, node_ref[...], tid, diw], axis=0)
    out_ref[0] = jnp.transpose(h)


def kernel(x, node_emb, time_in_day_emb, day_in_week_emb, W, b):
    xf = jnp.transpose(x, (1, 3, 0, 2)).reshape(R, N)
    nodeT = node_emb.T
    tidT = time_in_day_emb.T
    diwT = day_in_week_emb.T
    b2 = b.reshape(EDIM, 1)
    W3 = W.reshape(EDIM, L, C)
    Wbig = (W3[None, :, :, :, None] *
            jnp.eye(B, dtype=W.dtype)[:, None, None, None, :]).reshape(B, EDIM, R)

    out = pl.pallas_call(
        _stid_kernel,
        grid=(NUM_NB, B),
        in_specs=[
            pl.BlockSpec((R, BLOCK_N), lambda nb, bb: (0, nb)),
            pl.BlockSpec((1, EDIM, R), lambda nb, bb: (bb, 0, 0)),
            pl.BlockSpec((EDIM, 1), lambda nb, bb: (0, 0)),
            pl.BlockSpec((EDIM, BLOCK_N), lambda nb, bb: (0, nb)),
            pl.BlockSpec((EDIM, 288), lambda nb, bb: (0, 0)),
            pl.BlockSpec((EDIM, 7), lambda nb, bb: (0, 0)),
        ],
        out_specs=pl.BlockSpec((1, BLOCK_N, 4 * EDIM),
                               lambda nb, bb: (bb, nb, 0)),
        out_shape=jax.ShapeDtypeStruct((B, N, 4 * EDIM), jnp.float32),
    )(xf, Wbig, b2, nodeT, tidT, diwT)
    return jnp.transpose(out, (0, 2, 1))[..., None]


# speedup vs baseline: 5.2257x; 1.0334x over previous
import jax
import jax.numpy as jnp
from jax.experimental import pallas as pl

B, L, N, C = 8, 12, 50000, 3
EDIM = 32
R = L * C * B

BLOCK_N = 12800
NUM_NB = -(-N // BLOCK_N)
VW = 128


def _stid_kernel(xf_ref, wb_ref, b_ref, node_ref, tidT_ref, diwT_ref, out_ref):
    bb = pl.program_id(1)
    xfb = xf_ref[...]

    ts = jax.lax.dot_general(
        wb_ref[0], xfb, (((1,), (0,)), ((), ())),
        preferred_element_type=jnp.float32)
    ts = ts + b_ref[...]

    tvals = xf_ref[pl.ds((L - 1) * 24 + 8 + bb, 1), :]
    dvals = xf_ref[pl.ds((L - 1) * 24 + 16 + bb, 1), :]
    tid_idx = jnp.clip((tvals * 288.0).astype(jnp.int32), 0, 287)
    diw_idx = jnp.clip((dvals * 7.0).astype(jnp.int32), 0, 6)

    c0 = tidT_ref[:, 0:VW]
    c1 = tidT_ref[:, VW:2 * VW]
    c2 = tidT_ref[:, 2 * VW:288]
    dw = diwT_ref[...]
    tid_parts = []
    diw_parts = []
    for j in range(BLOCK_N // VW):
        tj = jnp.broadcast_to(tid_idx[:, VW * j:VW * (j + 1)], (EDIM, VW))
        g0 = jnp.take_along_axis(c0, jnp.clip(tj, 0, VW - 1), axis=1)
        g1 = jnp.take_along_axis(c1, jnp.clip(tj - VW, 0, VW - 1), axis=1)
        g2 = jnp.take_along_axis(c2, jnp.clip(tj - 2 * VW, 0, 31), axis=1)
        tid_parts.append(jnp.where(tj < VW, g0, jnp.where(tj < 2 * VW, g1, g2)))
        dj = jnp.broadcast_to(diw_idx[:, VW * j:VW * (j + 1)], (EDIM, VW))
        diw_parts.append(jnp.take_along_axis(dw, dj, axis=1))
    tid = jnp.concatenate(tid_parts, axis=1)
    diw = jnp.concatenate(diw_parts, axis=1)

    h = jnp.concatenate([ts, node_ref[...], tid, diw], axis=0)
    out_ref[0] = jnp.transpose(h)


def kernel(x, node_emb, time_in_day_emb, day_in_week_emb, W, b):
    xf = jnp.transpose(x, (1, 3, 0, 2)).reshape(R, N)
    nodeT = node_emb.T
    tidT = time_in_day_emb.T
    diwT = day_in_week_emb.T
    b2 = b.reshape(EDIM, 1)
    W3 = W.reshape(EDIM, L, C)
    Wbig = (W3[None, :, :, :, None] *
            jnp.eye(B, dtype=W.dtype)[:, None, None, None, :]).reshape(B, EDIM, R)

    out = pl.pallas_call(
        _stid_kernel,
        grid=(NUM_NB, B),
        in_specs=[
            pl.BlockSpec((R, BLOCK_N), lambda nb, bb: (0, nb)),
            pl.BlockSpec((1, EDIM, R), lambda nb, bb: (bb, 0, 0)),
            pl.BlockSpec((EDIM, 1), lambda nb, bb: (0, 0)),
            pl.BlockSpec((EDIM, BLOCK_N), lambda nb, bb: (0, nb)),
            pl.BlockSpec((EDIM, 288), lambda nb, bb: (0, 0)),
            pl.BlockSpec((EDIM, 7), lambda nb, bb: (0, 0)),
        ],
        out_specs=pl.BlockSpec((1, BLOCK_N, 4 * EDIM),
                               lambda nb, bb: (bb, nb, 0)),
        out_shape=jax.ShapeDtypeStruct((B, N, 4 * EDIM), jnp.float32),
    )(xf, Wbig, b2, nodeT, tidT, diwT)
    return jnp.transpose(out, (0, 2, 1))[..., None]
